# baseline (speedup 1.0000x reference)
import functools

import jax
import jax.numpy as jnp
from jax import lax
from jax.experimental import pallas as pl
from jax.experimental.pallas import tpu as pltpu
from jax.experimental.pallas import tpu_sc as plsc

_L = 16


@functools.lru_cache(maxsize=None)
def _make_permute_kernel(B: int, D: int):
    NC, NS = 2, 16
    NW = NC * NS
    assert B % NW == 0 and D % _L == 0
    RW = B // NW
    R = 128
    assert RW % R == 0
    n_chunks = RW // R
    NB = 3
    assert n_chunks >= NB
    JB = D // _L
    CW = R * D

    mesh = plsc.VectorSubcoreMesh(core_axis_name="c", subcore_axis_name="s")

    @functools.partial(
        pl.kernel,
        mesh=mesh,
        compiler_params=pltpu.CompilerParams(needs_layout_passes=False),
        out_type=[
            jax.ShapeDtypeStruct((B * D,), jnp.float32),
            jax.ShapeDtypeStruct((B,), jnp.float32),
        ],
        scratch_types=(
            [pltpu.VMEM((CW,), jnp.float32) for _ in range(2 * NB)]
            + [
                pltpu.VMEM((D,), jnp.int32),
                pltpu.VMEM((RW,), jnp.float32),
            ]
            + [pltpu.SemaphoreType.DMA for _ in range(3 * NB + 1)]
        ),
    )
    def permute_kernel(x_hbm, perm_hbm, y_hbm, ld_hbm,
                       in0, in1, in2, out0, out1, out2, perm_v, z_v,
                       is0, is1, is2, os0, os1, os2, ps0, ps1, ps2, zsem):
        wid = lax.axis_index("s") * NC + lax.axis_index("c")
        base = wid * (RW * D)
        ins, outs = (in0, in1, in2), (out0, out1, out2)
        isems, osems = (is0, is1, is2), (os0, os1, os2)
        psems = (ps0, ps1, ps2)
        H = CW // 2

        def in_copy(g, b):
            return pltpu.make_async_copy(
                x_hbm.at[pl.ds(base + g * CW, CW)], ins[b], isems[b])

        def out_copies(g, b):
            return (
                pltpu.make_async_copy(
                    outs[b].at[pl.ds(0, H)],
                    y_hbm.at[pl.ds(base + g * CW, H)], osems[b]),
                pltpu.make_async_copy(
                    outs[b].at[pl.ds(H, H)],
                    y_hbm.at[pl.ds(base + g * CW + H, H)], psems[b]),
            )

        for b in range(NB):
            in_copy(b, b).start()

        pltpu.sync_copy(perm_hbm, perm_v)
        pvecs = [perm_v[pl.ds(j * _L, _L)] for j in range(JB)]

        @plsc.parallel_loop(0, RW // _L, unroll=4)
        def _(i):
            z_v[pl.ds(i * _L, _L)] = jnp.zeros((_L,), jnp.float32)

        pltpu.make_async_copy(z_v, ld_hbm.at[pl.ds(wid * RW, RW)], zsem).start()

        for g in range(n_chunks):
            b = g % NB
            in_copy(g, b).wait()
            if g >= NB:
                for cp in out_copies(g - NB, b):
                    cp.wait()
            src, dst = ins[b], outs[b]

            @plsc.parallel_loop(0, R, unroll=8)
            def _(r):
                rb = r * D
                for j in range(JB):
                    val = plsc.load_gather(src, [pvecs[j] + rb])
                    dst[pl.ds(rb + j * _L, _L)] = val

            for cp in out_copies(g, b):
                cp.start()
            if g + NB < n_chunks:
                in_copy(g + NB, b).start()
        for g in range(n_chunks - NB, n_chunks):
            for cp in out_copies(g, g % NB):
                cp.wait()
        pltpu.make_async_copy(z_v, ld_hbm.at[pl.ds(wid * RW, RW)], zsem).wait()

    return permute_kernel


def kernel(x, perm):
    B, D = x.shape
    k = _make_permute_kernel(B, D)
    y_flat, log_det = k(x.reshape(B * D), perm.astype(jnp.int32))
    return y_flat.reshape(B, D), log_det


# speedup vs baseline: 1.0103x; 1.0103x over previous
import functools

import jax
import jax.numpy as jnp
from jax import lax
from jax.experimental import pallas as pl
from jax.experimental.pallas import tpu as pltpu
from jax.experimental.pallas import tpu_sc as plsc

_L = 16


@functools.lru_cache(maxsize=None)
def _make_permute_kernel(B: int, D: int):
    NC, NS = 2, 16
    NW = NC * NS
    assert B % NW == 0 and D % _L == 0
    RW = B // NW
    R = 128
    assert RW % R == 0
    n_chunks = RW // R
    NB = 3
    assert n_chunks >= NB
    JB = D // _L
    CW = R * D

    mesh = plsc.VectorSubcoreMesh(core_axis_name="c", subcore_axis_name="s")

    @functools.partial(
        pl.kernel,
        mesh=mesh,
        compiler_params=pltpu.CompilerParams(needs_layout_passes=False),
        out_type=[
            jax.ShapeDtypeStruct((B * D,), jnp.float32),
            jax.ShapeDtypeStruct((B,), jnp.float32),
        ],
        scratch_types=(
            [pltpu.VMEM((CW,), jnp.float32) for _ in range(2 * NB)]
            + [
                pltpu.VMEM((D,), jnp.int32),
                pltpu.VMEM((RW,), jnp.float32),
            ]
            + [pltpu.SemaphoreType.DMA for _ in range(2 * NB + 1)]
        ),
    )
    def permute_kernel(x_hbm, perm_hbm, y_hbm, ld_hbm,
                       in0, in1, in2, out0, out1, out2, perm_v, z_v,
                       is0, is1, is2, os0, os1, os2, zsem):
        wid = lax.axis_index("s") * NC + lax.axis_index("c")
        base = wid * (RW * D)
        ins, outs = (in0, in1, in2), (out0, out1, out2)
        isems, osems = (is0, is1, is2), (os0, os1, os2)

        def in_copy(g, b):
            return pltpu.make_async_copy(
                x_hbm.at[pl.ds(base + g * CW, CW)], ins[b], isems[b])

        def out_copy(g, b):
            return pltpu.make_async_copy(
                outs[b], y_hbm.at[pl.ds(base + g * CW, CW)], osems[b])

        for b in range(NB):
            in_copy(b, b).start()

        pltpu.sync_copy(perm_hbm, perm_v)
        pvecs = [perm_v[pl.ds(j * _L, _L)] for j in range(JB)]

        @plsc.parallel_loop(0, RW // _L, unroll=4)
        def _(i):
            z_v[pl.ds(i * _L, _L)] = jnp.zeros((_L,), jnp.float32)

        pltpu.make_async_copy(z_v, ld_hbm.at[pl.ds(wid * RW, RW)], zsem).start()

        for g in range(n_chunks):
            b = g % NB
            in_copy(g, b).wait()
            if g >= NB:
                out_copy(g - NB, b).wait()
            src, dst = ins[b], outs[b]

            @plsc.parallel_loop(0, R, unroll=8)
            def _(r):
                rb = r * D
                for j in range(JB):
                    val = plsc.load_gather(src, [pvecs[j] + rb])
                    dst[pl.ds(rb + j * _L, _L)] = val

            out_copy(g, b).start()
            if g + NB < n_chunks:
                in_copy(g + NB, b).start()
        for g in range(n_chunks - NB, n_chunks):
            out_copy(g, g % NB).wait()
        pltpu.make_async_copy(z_v, ld_hbm.at[pl.ds(wid * RW, RW)], zsem).wait()

    return permute_kernel


def kernel(x, perm):
    B, D = x.shape
    k = _make_permute_kernel(B, D)
    y_flat, log_det = k(x.reshape(B * D), perm.astype(jnp.int32))
    return y_flat.reshape(B, D), log_det
